# baseline (speedup 1.0000x reference)
import functools

import jax
import jax.numpy as jnp
from jax import lax
from jax.experimental import pallas as pl
from jax.experimental.pallas import tpu as pltpu
from jax.experimental.pallas import tpu_sc as plsc

N = 10000
NPAD = 10240
E = 160000
H = 768
D1 = 512
D2 = 256
EPS = 1e-6

ROW_BLK = 1000


NT = 16
NSC = 2
C = 80
NSTEP = 128
E_PAD = NT * NSTEP * C
SHARD = NSTEP * C
K = 8
NGRP = NSTEP // K


def _make_seg_sum(nch, with_den):
    cps = nch // NSC
    assert cps * NSC == nch

    out_type = [jax.ShapeDtypeStruct((nch, NPAD, 128), jnp.float32)]
    if with_den:
        out_type.append(jax.ShapeDtypeStruct((NPAD,), jnp.float32))

    scratch = dict(
        acc=pltpu.VMEM_SHARED((NPAD, 128), jnp.float32),
        src_st=pltpu.VMEM((K * C,), jnp.int32),
        w_st=pltpu.VMEM((K * C,), jnp.float32),
        we_st=pltpu.VMEM((K * C * 16,), jnp.float32),
        dst_st=pltpu.VMEM((K, C), jnp.int32),
        rows=pltpu.VMEM((C, 128), jnp.float32),
        rows1=pltpu.VMEM((C, 128), jnp.float32),
        gsem=pltpu.SemaphoreType.DMA,
        ssem=pltpu.SemaphoreType.DMA,
    )
    if with_den:
        scratch['den_acc'] = pltpu.VMEM_SHARED((NPAD,), jnp.float32)
        scratch['dzero'] = pltpu.VMEM((NPAD // NT,), jnp.float32)

    def body(*refs, acc, src_st, w_st, we_st, dst_st, rows, rows1, gsem,
             ssem, den_acc=None, dzero=None):
        y_refs = refs[:nch]
        src_hbm, dst_hbm, w_hbm, we_hbm, num_hbm = refs[nch:nch + 5]
        if with_den:
            den_hbm = refs[nch + 5]

        cid = lax.axis_index("c")
        sid = lax.axis_index("s")
        my_rows = NPAD // NT
        e0 = sid * SHARD

        def _zrow(j, _):
            for k in range(8):
                rows[j, pl.ds(k * 16, 16)] = jnp.zeros((16,), jnp.float32)
            return 0

        def _zero_my_acc():
            lax.fori_loop(0, C, _zrow, 0)
            for r in range(my_rows // C):
                pltpu.sync_copy(
                    rows, acc.at[pl.ds(sid * my_rows + r * C, C)])

        _zero_my_acc()
        if with_den:
            def _dz(j, _):
                dzero[pl.ds(j * 16, 16)] = jnp.zeros((16,), jnp.float32)
                return 0
            lax.fori_loop(0, my_rows // 16, _dz, 0)
            pltpu.sync_copy(dzero, den_acc.at[pl.ds(sid * my_rows, my_rows)])
        plsc.subcore_barrier()

        if with_den:
            @pl.when(cid == 0)
            def _():
                def den_grp(g, _):
                    pltpu.sync_copy(
                        w_hbm.at[pl.ds(e0 + g * K * C, K * C)], w_st)
                    pltpu.sync_copy(dst_hbm.at[sid, g], dst_st)

                    def den_step(s, _):
                        pltpu.sync_copy(w_st.at[pl.ds(s * C, C)],
                                        den_acc.at[dst_st.at[s]], add=True)
                        return 0
                    lax.fori_loop(0, K, den_step, 0)
                    return 0
                lax.fori_loop(0, NGRP, den_grp, 0)

        def _scale(buf, s):
            def row8(j8, _):
                for l in range(8):
                    j = j8 * 8 + l
                    wv = we_st[pl.ds((s * C + j) * 16, 16)]
                    for k in range(8):
                        sl = pl.ds(k * 16, 16)
                        buf[j, sl] = buf[j, sl] * wv
                return 0
            lax.fori_loop(0, C // 8, row8, 0)

        bufs = (rows, rows1)

        def edge_chunk(lc):
            def issue(s, buf):
                for ch in range(nch):
                    @pl.when((cid == ch // cps) & (lc == ch % cps))
                    def _(y_ref=y_refs[ch]):
                        pltpu.async_copy(
                            y_ref.at[src_st.at[pl.ds(s * C, C)]], buf, gsem)

            def grp(g, _):
                pltpu.sync_copy(src_hbm.at[pl.ds(e0 + g * K * C, K * C)],
                                src_st)
                pltpu.sync_copy(
                    we_hbm.at[pl.ds((e0 + g * K * C) * 16, K * C * 16)],
                    we_st)
                pltpu.sync_copy(dst_hbm.at[sid, g], dst_st)
                issue(0, bufs[0])
                scatters = []
                for s in range(K):
                    pltpu.make_async_copy(
                        y_refs[0].at[src_st.at[pl.ds(s * C, C)]],
                        bufs[s % 2], gsem).wait()
                    if s + 1 < K:
                        if s >= 1:
                            scatters[s - 1].wait()
                        issue(s + 1, bufs[(s + 1) % 2])
                    _scale(bufs[s % 2], s)
                    scatters.append(pltpu.async_copy(
                        bufs[s % 2], acc.at[dst_st.at[s]], ssem, add=True))
                scatters[K - 2].wait()
                scatters[K - 1].wait()
                return 0
            lax.fori_loop(0, NGRP, grp, 0)

        def chunk_pass(lc, _):
            edge_chunk(lc)
            plsc.subcore_barrier()
            q = cid * cps + lc
            pltpu.sync_copy(
                acc.at[pl.ds(sid * my_rows, my_rows)],
                num_hbm.at[q].at[pl.ds(sid * my_rows, my_rows)])
            _zero_my_acc()
            plsc.subcore_barrier()
            return 0
        lax.fori_loop(0, cps, chunk_pass, 0)

        if with_den:
            @pl.when(cid == 0)
            def _():
                pltpu.sync_copy(den_acc.at[pl.ds(sid * my_rows, my_rows)],
                                den_hbm.at[pl.ds(sid * my_rows, my_rows)])

    mesh = plsc.VectorSubcoreMesh(core_axis_name="c", subcore_axis_name="s")
    return pl.kernel(body, out_type=tuple(out_type), mesh=mesh,
                     scratch_types=scratch)


def _stage_a_body(ft_ref, pn_ref, tl_ref, wd_ref, bd_ref, wcat_ref, u_ref,
                  v_ref, bias_ref, y_ref):
    pooled = jnp.tanh(
        jnp.dot(ft_ref[...], wd_ref[...], preferred_element_type=jnp.float32)
        + bd_ref[...])
    y = jnp.dot(pooled, wcat_ref[...], preferred_element_type=jnp.float32)
    y = y + pn_ref[...] * u_ref[...] + tl_ref[...] * v_ref[...] + bias_ref[...]
    y_ref[...] = y


def _stage_a(ft, p_num, text_len, wd, bd, wcat, u, v, bias):
    grid = (N // ROW_BLK,)
    return pl.pallas_call(
        _stage_a_body,
        grid=grid,
        in_specs=[
            pl.BlockSpec((ROW_BLK, H), lambda i: (i, 0)),
            pl.BlockSpec((ROW_BLK, 1), lambda i: (i, 0)),
            pl.BlockSpec((ROW_BLK, 1), lambda i: (i, 0)),
            pl.BlockSpec((H, H), lambda i: (0, 0)),
            pl.BlockSpec((1, H), lambda i: (0, 0)),
            pl.BlockSpec((H, 2 * D1), lambda i: (0, 0)),
            pl.BlockSpec((1, 2 * D1), lambda i: (0, 0)),
            pl.BlockSpec((1, 2 * D1), lambda i: (0, 0)),
            pl.BlockSpec((1, 2 * D1), lambda i: (0, 0)),
        ],
        out_specs=pl.BlockSpec((ROW_BLK, 2 * D1), lambda i: (i, 0)),
        out_shape=jax.ShapeDtypeStruct((N, 2 * D1), jnp.float32),
    )(ft, p_num, text_len, wd, bd, wcat, u, v, bias)


def _stage_c_body(num_ref, den_ref, r_ref, w2_ref, bias2_ref, y_ref):
    agg = num_ref[...] / jnp.maximum(den_ref[...], EPS)
    h = jax.nn.relu(agg + r_ref[...])
    y_ref[...] = (
        jnp.dot(h, w2_ref[...], preferred_element_type=jnp.float32)
        + bias2_ref[...])


def _stage_c(num1, den, r1, w2cat, bias2):
    grid = (N // ROW_BLK,)
    return pl.pallas_call(
        _stage_c_body,
        grid=grid,
        in_specs=[
            pl.BlockSpec((ROW_BLK, D1), lambda i: (i, 0)),
            pl.BlockSpec((ROW_BLK, 1), lambda i: (i, 0)),
            pl.BlockSpec((ROW_BLK, D1), lambda i: (i, 0)),
            pl.BlockSpec((D1, 2 * D2), lambda i: (0, 0)),
            pl.BlockSpec((1, 2 * D2), lambda i: (0, 0)),
        ],
        out_specs=pl.BlockSpec((ROW_BLK, 2 * D2), lambda i: (i, 0)),
        out_shape=jax.ShapeDtypeStruct((N, 2 * D2), jnp.float32),
    )(num1, den, r1, w2cat, bias2)


def _stage_e_body(num_ref, den_ref, r_ref, y_ref):
    y_ref[...] = num_ref[...] / jnp.maximum(den_ref[...], EPS) + r_ref[...]


def _stage_e(num2, den, r2):
    grid = (N // ROW_BLK,)
    return pl.pallas_call(
        _stage_e_body,
        grid=grid,
        in_specs=[
            pl.BlockSpec((ROW_BLK, D2), lambda i: (i, 0)),
            pl.BlockSpec((ROW_BLK, 1), lambda i: (i, 0)),
            pl.BlockSpec((ROW_BLK, D2), lambda i: (i, 0)),
        ],
        out_specs=pl.BlockSpec((ROW_BLK, D2), lambda i: (i, 0)),
        out_shape=jax.ShapeDtypeStruct((N, D2), jnp.float32),
    )(num2, den, r2)


_seg1 = _make_seg_sum(D1 // 128, with_den=True)
_seg2 = _make_seg_sum(D2 // 128, with_den=False)


def kernel(first_token, p_num, text_len, edge_index, edge_weight, W_dense,
           b_dense, W1_l, W1_r, b1, W2_l, W2_r, b2):
    pad = E_PAD - E
    src = jnp.concatenate(
        [edge_index[0].astype(jnp.int32), jnp.zeros((pad,), jnp.int32)])
    dst2d = jnp.concatenate(
        [edge_index[1].astype(jnp.int32),
         jnp.full((pad,), N, jnp.int32)]).reshape(NT, NGRP, K, C)
    w_pad = jnp.concatenate([edge_weight, jnp.zeros((pad,), jnp.float32)])
    w_exp = jnp.broadcast_to(w_pad[:, None], (E_PAD, 16)).reshape(-1)

    wcat1 = jnp.concatenate([W1_l[:H], W1_r[:H]], axis=1)
    u1 = jnp.concatenate([W1_l[H], W1_r[H]])[None, :]
    v1 = jnp.concatenate([W1_l[H + 1], W1_r[H + 1]])[None, :]
    bias1 = jnp.concatenate([jnp.zeros((D1,), jnp.float32), b1])[None, :]
    w2cat = jnp.concatenate([W2_l, W2_r], axis=1)
    bias2 = jnp.concatenate([jnp.zeros((D2,), jnp.float32), b2])[None, :]

    y1r1 = _stage_a(first_token, p_num, text_len, W_dense, b_dense[None, :],
                    wcat1, u1, v1, bias1)
    y1, r1 = y1r1[:, :D1], y1r1[:, D1:]

    nch1 = D1 // 128
    y1c = y1.reshape(N, nch1, 128).transpose(1, 0, 2)
    num1c, den_pad = _seg1(*(y1c[c] for c in range(nch1)), src, dst2d,
                           w_pad, w_exp)
    num1 = num1c[:, :N].transpose(1, 0, 2).reshape(N, D1)
    den = den_pad[:N, None]

    y2r2 = _stage_c(num1, den, r1, w2cat, bias2)
    y2, r2 = y2r2[:, :D2], y2r2[:, D2:]

    nch2 = D2 // 128
    y2c = y2.reshape(N, nch2, 128).transpose(1, 0, 2)
    (num2c,) = _seg2(*(y2c[c] for c in range(nch2)), src, dst2d, w_pad,
                     w_exp)
    num2 = num2c[:, :N].transpose(1, 0, 2).reshape(N, D2)

    return _stage_e(num2, den, r2)


# speedup vs baseline: 1.1871x; 1.1871x over previous
import functools

import jax
import jax.numpy as jnp
from jax import lax
from jax.experimental import pallas as pl
from jax.experimental.pallas import tpu as pltpu
from jax.experimental.pallas import tpu_sc as plsc

N = 10000
NPAD = 10240
E = 160000
H = 768
D1 = 512
D2 = 256
EPS = 1e-6

ROW_BLK = 1000


NT = 16
NSC = 2
C = 80
NSTEP = 128
E_PAD = NT * NSTEP * C
SHARD = NSTEP * C
K = 8
NGRP = NSTEP // K


def _make_seg_sum(nch, with_den):
    cps = nch // NSC
    assert cps * NSC == nch

    out_type = [jax.ShapeDtypeStruct((nch, NPAD, 128), jnp.float32)]
    if with_den:
        out_type.append(jax.ShapeDtypeStruct((NPAD,), jnp.float32))

    scratch = dict(
        acc=pltpu.VMEM_SHARED((NPAD, 128), jnp.float32),
        src_st=pltpu.VMEM((K * C,), jnp.int32),
        w_st=pltpu.VMEM((K * C,), jnp.float32),
        dst_st=pltpu.VMEM((K, C), jnp.int32),
        rows=pltpu.VMEM((C, 128), jnp.float32),
        rows1=pltpu.VMEM((C, 128), jnp.float32),
        gsem=pltpu.SemaphoreType.DMA,
        ssem=pltpu.SemaphoreType.DMA,
    )
    if with_den:
        scratch['den_acc'] = pltpu.VMEM_SHARED((NPAD,), jnp.float32)
        scratch['dzero'] = pltpu.VMEM((NPAD // NT,), jnp.float32)

    def body(*refs, acc, src_st, w_st, dst_st, rows, rows1, gsem,
             ssem, den_acc=None, dzero=None):
        y_refs = refs[:nch]
        src_hbm, dst_hbm, w_hbm, num_hbm = refs[nch:nch + 4]
        if with_den:
            den_hbm = refs[nch + 4]

        cid = lax.axis_index("c")
        sid = lax.axis_index("s")
        my_rows = NPAD // NT
        e0 = sid * SHARD

        def _zrow(j, _):
            for k in range(8):
                rows[j, pl.ds(k * 16, 16)] = jnp.zeros((16,), jnp.float32)
            return 0

        def _zero_my_acc():
            lax.fori_loop(0, C, _zrow, 0)
            for r in range(my_rows // C):
                pltpu.sync_copy(
                    rows, acc.at[pl.ds(sid * my_rows + r * C, C)])

        _zero_my_acc()
        if with_den:
            def _dz(j, _):
                dzero[pl.ds(j * 16, 16)] = jnp.zeros((16,), jnp.float32)
                return 0
            lax.fori_loop(0, my_rows // 16, _dz, 0)
            pltpu.sync_copy(dzero, den_acc.at[pl.ds(sid * my_rows, my_rows)])
        plsc.subcore_barrier()

        if with_den:
            @pl.when(cid == 0)
            def _():
                def den_grp(g, _):
                    pltpu.sync_copy(
                        w_hbm.at[pl.ds(e0 + g * K * C, K * C)], w_st)
                    pltpu.sync_copy(dst_hbm.at[sid, g], dst_st)

                    def den_step(s, _):
                        pltpu.sync_copy(w_st.at[pl.ds(s * C, C)],
                                        den_acc.at[dst_st.at[s]], add=True)
                        return 0
                    lax.fori_loop(0, K, den_step, 0)
                    return 0
                lax.fori_loop(0, NGRP, den_grp, 0)

        def _scale(buf, s):
            def row16(j16, _):
                wvec = w_st[pl.ds(s * C + j16 * 16, 16)]
                for l in range(16):
                    wv = jnp.full((16,), wvec[l], jnp.float32)
                    j = j16 * 16 + l
                    for k in range(8):
                        sl = pl.ds(k * 16, 16)
                        buf[j, sl] = buf[j, sl] * wv
                return 0
            lax.fori_loop(0, C // 16, row16, 0)

        bufs = (rows, rows1)

        def edge_chunk(lc):
            def issue(s, buf):
                for ch in range(nch):
                    @pl.when((cid == ch // cps) & (lc == ch % cps))
                    def _(y_ref=y_refs[ch]):
                        pltpu.async_copy(
                            y_ref.at[src_st.at[pl.ds(s * C, C)]], buf, gsem)

            def grp(g, _):
                pltpu.sync_copy(src_hbm.at[pl.ds(e0 + g * K * C, K * C)],
                                src_st)
                pltpu.sync_copy(w_hbm.at[pl.ds(e0 + g * K * C, K * C)], w_st)
                pltpu.sync_copy(dst_hbm.at[sid, g], dst_st)
                issue(0, bufs[0])
                scatters = []
                for s in range(K):
                    pltpu.make_async_copy(
                        y_refs[0].at[src_st.at[pl.ds(s * C, C)]],
                        bufs[s % 2], gsem).wait()
                    if s + 1 < K:
                        if s >= 1:
                            scatters[s - 1].wait()
                        issue(s + 1, bufs[(s + 1) % 2])
                    _scale(bufs[s % 2], s)
                    scatters.append(pltpu.async_copy(
                        bufs[s % 2], acc.at[dst_st.at[s]], ssem, add=True))
                scatters[K - 2].wait()
                scatters[K - 1].wait()
                return 0
            lax.fori_loop(0, NGRP, grp, 0)

        def chunk_pass(lc, _):
            edge_chunk(lc)
            plsc.subcore_barrier()
            q = cid * cps + lc
            pltpu.sync_copy(
                acc.at[pl.ds(sid * my_rows, my_rows)],
                num_hbm.at[q].at[pl.ds(sid * my_rows, my_rows)])
            _zero_my_acc()
            plsc.subcore_barrier()
            return 0
        lax.fori_loop(0, cps, chunk_pass, 0)

        if with_den:
            @pl.when(cid == 0)
            def _():
                pltpu.sync_copy(den_acc.at[pl.ds(sid * my_rows, my_rows)],
                                den_hbm.at[pl.ds(sid * my_rows, my_rows)])

    mesh = plsc.VectorSubcoreMesh(core_axis_name="c", subcore_axis_name="s")
    return pl.kernel(body, out_type=tuple(out_type), mesh=mesh,
                     scratch_types=scratch)


def _stage_a_body(ft_ref, pn_ref, tl_ref, wd_ref, bd_ref, wcat_ref, u_ref,
                  v_ref, bias_ref, y_ref):
    pooled = jnp.tanh(
        jnp.dot(ft_ref[...], wd_ref[...], preferred_element_type=jnp.float32)
        + bd_ref[...])
    y = jnp.dot(pooled, wcat_ref[...], preferred_element_type=jnp.float32)
    y = y + pn_ref[...] * u_ref[...] + tl_ref[...] * v_ref[...] + bias_ref[...]
    y_ref[...] = y


def _stage_a(ft, p_num, text_len, wd, bd, wcat, u, v, bias):
    grid = (N // ROW_BLK,)
    return pl.pallas_call(
        _stage_a_body,
        grid=grid,
        in_specs=[
            pl.BlockSpec((ROW_BLK, H), lambda i: (i, 0)),
            pl.BlockSpec((ROW_BLK, 1), lambda i: (i, 0)),
            pl.BlockSpec((ROW_BLK, 1), lambda i: (i, 0)),
            pl.BlockSpec((H, H), lambda i: (0, 0)),
            pl.BlockSpec((1, H), lambda i: (0, 0)),
            pl.BlockSpec((H, 2 * D1), lambda i: (0, 0)),
            pl.BlockSpec((1, 2 * D1), lambda i: (0, 0)),
            pl.BlockSpec((1, 2 * D1), lambda i: (0, 0)),
            pl.BlockSpec((1, 2 * D1), lambda i: (0, 0)),
        ],
        out_specs=pl.BlockSpec((ROW_BLK, 2 * D1), lambda i: (i, 0)),
        out_shape=jax.ShapeDtypeStruct((N, 2 * D1), jnp.float32),
    )(ft, p_num, text_len, wd, bd, wcat, u, v, bias)


def _stage_c_body(num_ref, den_ref, r_ref, w2_ref, bias2_ref, y_ref):
    agg = num_ref[...] / jnp.maximum(den_ref[...], EPS)
    h = jax.nn.relu(agg + r_ref[...])
    y_ref[...] = (
        jnp.dot(h, w2_ref[...], preferred_element_type=jnp.float32)
        + bias2_ref[...])


def _stage_c(num1, den, r1, w2cat, bias2):
    grid = (N // ROW_BLK,)
    return pl.pallas_call(
        _stage_c_body,
        grid=grid,
        in_specs=[
            pl.BlockSpec((ROW_BLK, D1), lambda i: (i, 0)),
            pl.BlockSpec((ROW_BLK, 1), lambda i: (i, 0)),
            pl.BlockSpec((ROW_BLK, D1), lambda i: (i, 0)),
            pl.BlockSpec((D1, 2 * D2), lambda i: (0, 0)),
            pl.BlockSpec((1, 2 * D2), lambda i: (0, 0)),
        ],
        out_specs=pl.BlockSpec((ROW_BLK, 2 * D2), lambda i: (i, 0)),
        out_shape=jax.ShapeDtypeStruct((N, 2 * D2), jnp.float32),
    )(num1, den, r1, w2cat, bias2)


def _stage_e_body(num_ref, den_ref, r_ref, y_ref):
    y_ref[...] = num_ref[...] / jnp.maximum(den_ref[...], EPS) + r_ref[...]


def _stage_e(num2, den, r2):
    grid = (N // ROW_BLK,)
    return pl.pallas_call(
        _stage_e_body,
        grid=grid,
        in_specs=[
            pl.BlockSpec((ROW_BLK, D2), lambda i: (i, 0)),
            pl.BlockSpec((ROW_BLK, 1), lambda i: (i, 0)),
            pl.BlockSpec((ROW_BLK, D2), lambda i: (i, 0)),
        ],
        out_specs=pl.BlockSpec((ROW_BLK, D2), lambda i: (i, 0)),
        out_shape=jax.ShapeDtypeStruct((N, D2), jnp.float32),
    )(num2, den, r2)


_seg1 = _make_seg_sum(D1 // 128, with_den=True)
_seg2 = _make_seg_sum(D2 // 128, with_den=False)


def kernel(first_token, p_num, text_len, edge_index, edge_weight, W_dense,
           b_dense, W1_l, W1_r, b1, W2_l, W2_r, b2):
    pad = E_PAD - E
    src = jnp.concatenate(
        [edge_index[0].astype(jnp.int32), jnp.zeros((pad,), jnp.int32)])
    dst2d = jnp.concatenate(
        [edge_index[1].astype(jnp.int32),
         jnp.full((pad,), N, jnp.int32)]).reshape(NT, NGRP, K, C)
    w_pad = jnp.concatenate([edge_weight, jnp.zeros((pad,), jnp.float32)])

    wcat1 = jnp.concatenate([W1_l[:H], W1_r[:H]], axis=1)
    u1 = jnp.concatenate([W1_l[H], W1_r[H]])[None, :]
    v1 = jnp.concatenate([W1_l[H + 1], W1_r[H + 1]])[None, :]
    bias1 = jnp.concatenate([jnp.zeros((D1,), jnp.float32), b1])[None, :]
    w2cat = jnp.concatenate([W2_l, W2_r], axis=1)
    bias2 = jnp.concatenate([jnp.zeros((D2,), jnp.float32), b2])[None, :]

    y1r1 = _stage_a(first_token, p_num, text_len, W_dense, b_dense[None, :],
                    wcat1, u1, v1, bias1)
    y1, r1 = y1r1[:, :D1], y1r1[:, D1:]

    nch1 = D1 // 128
    y1c = y1.reshape(N, nch1, 128).transpose(1, 0, 2)
    num1c, den_pad = _seg1(*(y1c[c] for c in range(nch1)), src, dst2d,
                           w_pad)
    num1 = num1c[:, :N].transpose(1, 0, 2).reshape(N, D1)
    den = den_pad[:N, None]

    y2r2 = _stage_c(num1, den, r1, w2cat, bias2)
    y2, r2 = y2r2[:, :D2], y2r2[:, D2:]

    nch2 = D2 // 128
    y2c = y2.reshape(N, nch2, 128).transpose(1, 0, 2)
    (num2c,) = _seg2(*(y2c[c] for c in range(nch2)), src, dst2d, w_pad)
    num2 = num2c[:, :N].transpose(1, 0, 2).reshape(N, D2)

    return _stage_e(num2, den, r2)


# speedup vs baseline: 1.3241x; 1.1154x over previous
import functools

import jax
import jax.numpy as jnp
from jax import lax
from jax.experimental import pallas as pl
from jax.experimental.pallas import tpu as pltpu
from jax.experimental.pallas import tpu_sc as plsc

N = 10000
NPAD = 10240
E = 160000
H = 768
D1 = 512
D2 = 256
EPS = 1e-6

ROW_BLK = 1000


NT = 16
NSC = 2
C = 80
NSTEP = 128
E_PAD = NT * NSTEP * C
SHARD = NSTEP * C
K = 16
NGRP = NSTEP // K


def _make_seg_sum(nch, with_den):
    cps = nch // NSC
    assert cps * NSC == nch

    out_type = [jax.ShapeDtypeStruct((nch, NPAD, 128), jnp.float32)]
    if with_den:
        out_type.append(jax.ShapeDtypeStruct((NPAD,), jnp.float32))

    scratch = dict(
        acc=pltpu.VMEM_SHARED((NPAD, 128), jnp.float32),
        src_st=pltpu.VMEM((K * C,), jnp.int32),
        w_st=pltpu.VMEM((K * C,), jnp.float32),
        dst_st=pltpu.VMEM((K, C), jnp.int32),
        rows=pltpu.VMEM((C, 128), jnp.float32),
        rows1=pltpu.VMEM((C, 128), jnp.float32),
        rows2=pltpu.VMEM((C, 128), jnp.float32),
        gsem=pltpu.SemaphoreType.DMA,
        ssem=pltpu.SemaphoreType.DMA,
    )
    if with_den:
        scratch['den_acc'] = pltpu.VMEM_SHARED((NPAD,), jnp.float32)
        scratch['dzero'] = pltpu.VMEM((NPAD // NT,), jnp.float32)

    def body(*refs, acc, src_st, w_st, dst_st, rows, rows1, rows2, gsem,
             ssem, den_acc=None, dzero=None):
        y_refs = refs[:nch]
        src_hbm, dst_hbm, w_hbm, num_hbm = refs[nch:nch + 4]
        if with_den:
            den_hbm = refs[nch + 4]

        cid = lax.axis_index("c")
        sid = lax.axis_index("s")
        my_rows = NPAD // NT
        e0 = sid * SHARD

        def _zrow(j, _):
            for k in range(8):
                rows[j, pl.ds(k * 16, 16)] = jnp.zeros((16,), jnp.float32)
            return 0

        def _zero_my_acc():
            lax.fori_loop(0, C, _zrow, 0)
            for r in range(my_rows // C):
                pltpu.sync_copy(
                    rows, acc.at[pl.ds(sid * my_rows + r * C, C)])

        _zero_my_acc()
        if with_den:
            def _dz(j, _):
                dzero[pl.ds(j * 16, 16)] = jnp.zeros((16,), jnp.float32)
                return 0
            lax.fori_loop(0, my_rows // 16, _dz, 0)
            pltpu.sync_copy(dzero, den_acc.at[pl.ds(sid * my_rows, my_rows)])
        plsc.subcore_barrier()

        if with_den:
            @pl.when(cid == 0)
            def _():
                def den_grp(g, _):
                    pltpu.sync_copy(
                        w_hbm.at[pl.ds(e0 + g * K * C, K * C)], w_st)
                    pltpu.sync_copy(dst_hbm.at[sid, g], dst_st)

                    def den_step(s, _):
                        pltpu.sync_copy(w_st.at[pl.ds(s * C, C)],
                                        den_acc.at[dst_st.at[s]], add=True)
                        return 0
                    lax.fori_loop(0, K, den_step, 0)
                    return 0
                lax.fori_loop(0, NGRP, den_grp, 0)

        def _scale(buf, s):
            def row16(j16, _):
                wvec = w_st[pl.ds(s * C + j16 * 16, 16)]
                for l in range(16):
                    wv = jnp.full((16,), wvec[l], jnp.float32)
                    j = j16 * 16 + l
                    for k in range(8):
                        sl = pl.ds(k * 16, 16)
                        buf[j, sl] = buf[j, sl] * wv
                return 0
            lax.fori_loop(0, C // 16, row16, 0)

        bufs = (rows, rows1, rows2)

        def edge_chunk(lc):
            def issue(s, buf):
                for ch in range(nch):
                    @pl.when((cid == ch // cps) & (lc == ch % cps))
                    def _(y_ref=y_refs[ch]):
                        pltpu.async_copy(
                            y_ref.at[src_st.at[pl.ds(s * C, C)]], buf, gsem)

            def grp(g, _):
                pltpu.sync_copy(src_hbm.at[pl.ds(e0 + g * K * C, K * C)],
                                src_st)
                pltpu.sync_copy(w_hbm.at[pl.ds(e0 + g * K * C, K * C)], w_st)
                pltpu.sync_copy(dst_hbm.at[sid, g], dst_st)
                issue(0, bufs[0])
                issue(1, bufs[1])
                scatters = []
                for s in range(K):
                    pltpu.make_async_copy(
                        y_refs[0].at[src_st.at[pl.ds(s * C, C)]],
                        bufs[s % 3], gsem).wait()
                    if s + 2 < K:
                        if s >= 1:
                            scatters[s - 1].wait()
                        issue(s + 2, bufs[(s + 2) % 3])
                    _scale(bufs[s % 3], s)
                    scatters.append(pltpu.async_copy(
                        bufs[s % 3], acc.at[dst_st.at[s]], ssem, add=True))
                scatters[K - 3].wait()
                scatters[K - 2].wait()
                scatters[K - 1].wait()
                return 0
            lax.fori_loop(0, NGRP, grp, 0)

        def chunk_pass(lc, _):
            edge_chunk(lc)
            plsc.subcore_barrier()
            q = cid * cps + lc
            pltpu.sync_copy(
                acc.at[pl.ds(sid * my_rows, my_rows)],
                num_hbm.at[q].at[pl.ds(sid * my_rows, my_rows)])
            _zero_my_acc()
            plsc.subcore_barrier()
            return 0
        lax.fori_loop(0, cps, chunk_pass, 0)

        if with_den:
            @pl.when(cid == 0)
            def _():
                pltpu.sync_copy(den_acc.at[pl.ds(sid * my_rows, my_rows)],
                                den_hbm.at[pl.ds(sid * my_rows, my_rows)])

    mesh = plsc.VectorSubcoreMesh(core_axis_name="c", subcore_axis_name="s")
    return pl.kernel(body, out_type=tuple(out_type), mesh=mesh,
                     scratch_types=scratch)


def _stage_a_body(ft_ref, pn_ref, tl_ref, wd_ref, bd_ref, wcat_ref, u_ref,
                  v_ref, bias_ref, y_ref):
    pooled = jnp.tanh(
        jnp.dot(ft_ref[...], wd_ref[...], preferred_element_type=jnp.float32)
        + bd_ref[...])
    y = jnp.dot(pooled, wcat_ref[...], preferred_element_type=jnp.float32)
    y = y + pn_ref[...] * u_ref[...] + tl_ref[...] * v_ref[...] + bias_ref[...]
    y_ref[...] = y


def _stage_a(ft, p_num, text_len, wd, bd, wcat, u, v, bias):
    grid = (N // ROW_BLK,)
    return pl.pallas_call(
        _stage_a_body,
        grid=grid,
        in_specs=[
            pl.BlockSpec((ROW_BLK, H), lambda i: (i, 0)),
            pl.BlockSpec((ROW_BLK, 1), lambda i: (i, 0)),
            pl.BlockSpec((ROW_BLK, 1), lambda i: (i, 0)),
            pl.BlockSpec((H, H), lambda i: (0, 0)),
            pl.BlockSpec((1, H), lambda i: (0, 0)),
            pl.BlockSpec((H, 2 * D1), lambda i: (0, 0)),
            pl.BlockSpec((1, 2 * D1), lambda i: (0, 0)),
            pl.BlockSpec((1, 2 * D1), lambda i: (0, 0)),
            pl.BlockSpec((1, 2 * D1), lambda i: (0, 0)),
        ],
        out_specs=pl.BlockSpec((ROW_BLK, 2 * D1), lambda i: (i, 0)),
        out_shape=jax.ShapeDtypeStruct((N, 2 * D1), jnp.float32),
    )(ft, p_num, text_len, wd, bd, wcat, u, v, bias)


def _stage_c_body(num_ref, den_ref, r_ref, w2_ref, bias2_ref, y_ref):
    agg = num_ref[...] / jnp.maximum(den_ref[...], EPS)
    h = jax.nn.relu(agg + r_ref[...])
    y_ref[...] = (
        jnp.dot(h, w2_ref[...], preferred_element_type=jnp.float32)
        + bias2_ref[...])


def _stage_c(num1, den, r1, w2cat, bias2):
    grid = (N // ROW_BLK,)
    return pl.pallas_call(
        _stage_c_body,
        grid=grid,
        in_specs=[
            pl.BlockSpec((ROW_BLK, D1), lambda i: (i, 0)),
            pl.BlockSpec((ROW_BLK, 1), lambda i: (i, 0)),
            pl.BlockSpec((ROW_BLK, D1), lambda i: (i, 0)),
            pl.BlockSpec((D1, 2 * D2), lambda i: (0, 0)),
            pl.BlockSpec((1, 2 * D2), lambda i: (0, 0)),
        ],
        out_specs=pl.BlockSpec((ROW_BLK, 2 * D2), lambda i: (i, 0)),
        out_shape=jax.ShapeDtypeStruct((N, 2 * D2), jnp.float32),
    )(num1, den, r1, w2cat, bias2)


def _stage_e_body(num_ref, den_ref, r_ref, y_ref):
    y_ref[...] = num_ref[...] / jnp.maximum(den_ref[...], EPS) + r_ref[...]


def _stage_e(num2, den, r2):
    grid = (N // ROW_BLK,)
    return pl.pallas_call(
        _stage_e_body,
        grid=grid,
        in_specs=[
            pl.BlockSpec((ROW_BLK, D2), lambda i: (i, 0)),
            pl.BlockSpec((ROW_BLK, 1), lambda i: (i, 0)),
            pl.BlockSpec((ROW_BLK, D2), lambda i: (i, 0)),
        ],
        out_specs=pl.BlockSpec((ROW_BLK, D2), lambda i: (i, 0)),
        out_shape=jax.ShapeDtypeStruct((N, D2), jnp.float32),
    )(num2, den, r2)


_seg1 = _make_seg_sum(D1 // 128, with_den=True)
_seg2 = _make_seg_sum(D2 // 128, with_den=False)


def kernel(first_token, p_num, text_len, edge_index, edge_weight, W_dense,
           b_dense, W1_l, W1_r, b1, W2_l, W2_r, b2):
    pad = E_PAD - E
    src = jnp.concatenate(
        [edge_index[0].astype(jnp.int32), jnp.zeros((pad,), jnp.int32)])
    dst2d = jnp.concatenate(
        [edge_index[1].astype(jnp.int32),
         jnp.full((pad,), N, jnp.int32)]).reshape(NT, NGRP, K, C)
    w_pad = jnp.concatenate([edge_weight, jnp.zeros((pad,), jnp.float32)])

    wcat1 = jnp.concatenate([W1_l[:H], W1_r[:H]], axis=1)
    u1 = jnp.concatenate([W1_l[H], W1_r[H]])[None, :]
    v1 = jnp.concatenate([W1_l[H + 1], W1_r[H + 1]])[None, :]
    bias1 = jnp.concatenate([jnp.zeros((D1,), jnp.float32), b1])[None, :]
    w2cat = jnp.concatenate([W2_l, W2_r], axis=1)
    bias2 = jnp.concatenate([jnp.zeros((D2,), jnp.float32), b2])[None, :]

    y1r1 = _stage_a(first_token, p_num, text_len, W_dense, b_dense[None, :],
                    wcat1, u1, v1, bias1)
    y1, r1 = y1r1[:, :D1], y1r1[:, D1:]

    nch1 = D1 // 128
    y1c = y1.reshape(N, nch1, 128).transpose(1, 0, 2)
    num1c, den_pad = _seg1(*(y1c[c] for c in range(nch1)), src, dst2d,
                           w_pad)
    num1 = num1c[:, :N].transpose(1, 0, 2).reshape(N, D1)
    den = den_pad[:N, None]

    y2r2 = _stage_c(num1, den, r1, w2cat, bias2)
    y2, r2 = y2r2[:, :D2], y2r2[:, D2:]

    nch2 = D2 // 128
    y2c = y2.reshape(N, nch2, 128).transpose(1, 0, 2)
    (num2c,) = _seg2(*(y2c[c] for c in range(nch2)), src, dst2d, w_pad)
    num2 = num2c[:, :N].transpose(1, 0, 2).reshape(N, D2)

    return _stage_e(num2, den, r2)
